# baseline (speedup 1.0000x reference)
import functools

import jax
import jax.numpy as jnp
from jax import lax
from jax.experimental import pallas as pl
from jax.experimental.pallas import tpu as pltpu
from jax.experimental.pallas import tpu_sc as plsc

EMB = 64
NC = 2
NS = 16
NW = NC * NS
BLK = 128
NBUF = 4

_mesh = plsc.VectorSubcoreMesh(core_axis_name="c", subcore_axis_name="s")


def _make_relayout(vocab: int):
    nblk = vocab // BLK
    tail = vocab - nblk * BLK

    @functools.partial(
        pl.kernel,
        mesh=_mesh,
        out_type=jax.ShapeDtypeStruct((vocab * EMB,), jnp.float32),
        scratch_types=[
            pltpu.VMEM((NBUF, EMB, BLK + 1), jnp.float32),
            pltpu.VMEM((NBUF, BLK * EMB), jnp.float32),
            pltpu.SemaphoreType.DMA((NBUF,)),
            pltpu.SemaphoreType.DMA((NBUF,)),
        ],
        compiler_params=pltpu.CompilerParams(needs_layout_passes=False),
    )
    def relayout_kernel(tab_hbm, tail_hbm, out_hbm, rbufs, wbufs, gsems, osems):
        wid = lax.axis_index("s") * NC + lax.axis_index("c")
        lanes = lax.iota(jnp.int32, 16)
        eq = [lanes + q * 16 for q in range(EMB // 16)]

        n_k = (nblk - wid + NW - 1) // NW

        def transpose(b, bvec, ncols):
            @plsc.parallel_loop(0, ncols, unroll=8)
            def _t(c):
                cvec = jnp.full((16,), c, jnp.int32)
                for q in range(EMB // 16):
                    v = plsc.load_gather(rbufs, [bvec, eq[q], cvec])
                    wbufs[b, pl.ds(c * EMB + q * 16, 16)] = v

        if tail:
            @pl.when(wid == NW - 1)
            def _tail():
                pltpu.sync_copy(tail_hbm, wbufs.at[0, pl.ds(0, tail * EMB)])
                pltpu.sync_copy(
                    wbufs.at[0, pl.ds(0, tail * EMB)],
                    out_hbm.at[pl.ds(nblk * BLK * EMB, tail * EMB)],
                )

        for b in range(NBUF):
            pltpu.async_copy(
                tab_hbm.at[:, pl.ds((wid + NW * b) * BLK, BLK)],
                rbufs.at[b, :, pl.ds(0, BLK)],
                gsems.at[b],
            )

        def step(k, _):
            b = lax.rem(k, NBUF)
            blk = wid + NW * k
            bvec = jnp.full((16,), b, jnp.int32)

            @pl.when(k >= NBUF)
            def _wait_out():
                pltpu.make_async_copy(
                    wbufs.at[b], out_hbm.at[pl.ds(0, BLK * EMB)], osems.at[b]
                ).wait()

            pltpu.make_async_copy(
                tab_hbm.at[:, pl.ds(blk * BLK, BLK)],
                rbufs.at[b, :, pl.ds(0, BLK)],
                gsems.at[b],
            ).wait()
            transpose(b, bvec, BLK)
            pltpu.async_copy(
                wbufs.at[b], out_hbm.at[pl.ds(blk * BLK * EMB, BLK * EMB)],
                osems.at[b],
            )

            @pl.when(k + NBUF < n_k)
            def _refill():
                nblk2 = wid + NW * (k + NBUF)
                pltpu.async_copy(
                    tab_hbm.at[:, pl.ds(nblk2 * BLK, BLK)],
                    rbufs.at[b, :, pl.ds(0, BLK)],
                    gsems.at[b],
                )

            return _

        lax.fori_loop(0, n_k, step, None)

        for b in range(NBUF):
            pltpu.make_async_copy(
                wbufs.at[b], out_hbm.at[pl.ds(0, BLK * EMB)], osems.at[b]
            ).wait()

    return relayout_kernel


def _make_gather(seq: int):
    et = EMB // 8

    @functools.partial(
        pl.kernel,
        mesh=_mesh,
        out_type=jax.ShapeDtypeStruct((seq, et, NW, 8, BLK), jnp.float32),
        scratch_types=[
            pltpu.VMEM((seq, BLK), jnp.int32),
            pltpu.VMEM((NBUF, BLK, EMB), jnp.float32),
            pltpu.VMEM((NBUF, et, 8, BLK + 1), jnp.float32),
            pltpu.SemaphoreType.DMA((NBUF,)),
            pltpu.SemaphoreType.DMA((NBUF,)),
        ],
        compiler_params=pltpu.CompilerParams(
            use_tc_tiling_on_sc=False, needs_layout_passes=False
        ),
    )
    def gather_kernel(ids_hbm, table_hbm, out_hbm, idx_v, rbufs, tbufs, gsems, osems):
        wid = lax.axis_index("s") * NC + lax.axis_index("c")

        pltpu.sync_copy(ids_hbm.at[:, pl.ds(wid * BLK, BLK)], idx_v)

        lanes = lax.iota(jnp.int32, 16)
        e8vec = [(lanes + q * 16) // 8 for q in range(EMB // 16)]
        elvec = [(lanes + q * 16) % 8 for q in range(EMB // 16)]

        for b in range(NBUF):
            pltpu.async_copy(table_hbm.at[idx_v.at[b]], rbufs.at[b], gsems.at[b])

        def step(i, _):
            b = lax.rem(i, NBUF)

            @pl.when(i >= NBUF)
            def _wait_out():
                pltpu.make_async_copy(
                    tbufs.at[b, :, :, pl.ds(0, BLK)],
                    out_hbm.at[0, :, wid],
                    osems.at[b],
                ).wait()

            pltpu.make_async_copy(
                table_hbm.at[idx_v.at[i]], rbufs.at[b], gsems.at[b]
            ).wait()

            bvec = jnp.full((16,), b, jnp.int32)

            @plsc.parallel_loop(0, BLK, unroll=8)
            def _transpose(r):
                rvec = jnp.full((16,), r, jnp.int32)
                for q in range(EMB // 16):
                    v = rbufs[b, r, pl.ds(q * 16, 16)]
                    plsc.store_scatter(tbufs, [bvec, e8vec[q], elvec[q], rvec], v)

            pltpu.async_copy(
                tbufs.at[b, :, :, pl.ds(0, BLK)], out_hbm.at[i, :, wid], osems.at[b]
            )

            @pl.when(i + NBUF < seq)
            def _refill():
                pltpu.async_copy(
                    table_hbm.at[idx_v.at[i + NBUF]], rbufs.at[b], gsems.at[b]
                )

            return _

        lax.fori_loop(0, seq, step, None)

        for b in range(NBUF):
            pltpu.make_async_copy(
                tbufs.at[b, :, :, pl.ds(0, BLK)],
                out_hbm.at[0, :, wid],
                osems.at[b],
            ).wait()

    return gather_kernel


def kernel(input_ids, token_table):
    batch, seq = input_ids.shape
    vocab, _ = token_table.shape
    ids_t = jnp.transpose(input_ids.astype(jnp.int32))
    table_t = jnp.transpose(token_table)
    nblk = vocab // BLK
    tail_flat = jnp.reshape(
        lax.slice(token_table, (nblk * BLK, 0), (vocab, EMB)),
        ((vocab - nblk * BLK) * EMB,),
    )
    table_lin = jnp.reshape(
        _make_relayout(vocab)(table_t, tail_flat), (vocab, EMB)
    )
    out5 = _make_gather(seq)(ids_t, table_lin)
    return jnp.transpose(out5, (2, 4, 0, 1, 3)).reshape(batch, seq, EMB)


# speedup vs baseline: 1.6382x; 1.6382x over previous
import functools

import jax
import jax.numpy as jnp
from jax import lax
from jax.experimental import pallas as pl
from jax.experimental.pallas import tpu as pltpu
from jax.experimental.pallas import tpu_sc as plsc

EMB = 64
NC = 2
NS = 16
NW = NC * NS
BLK = 128
NBUF = 4

_mesh = plsc.VectorSubcoreMesh(core_axis_name="c", subcore_axis_name="s")


def _make_relayout(vocab: int):
    nblk = vocab // BLK
    tail = vocab - nblk * BLK

    @functools.partial(
        pl.kernel,
        mesh=_mesh,
        out_type=jax.ShapeDtypeStruct((vocab * EMB,), jnp.float32),
        scratch_types=[
            pltpu.VMEM((NBUF, EMB, BLK), jnp.float32),
            pltpu.VMEM((NBUF, BLK * (EMB + 1)), jnp.float32),
            pltpu.VMEM((NBUF, BLK * EMB), jnp.float32),
            pltpu.SemaphoreType.DMA((NBUF,)),
            pltpu.SemaphoreType.DMA((NBUF,)),
        ],
        compiler_params=pltpu.CompilerParams(needs_layout_passes=False),
    )
    def relayout_kernel(tab_hbm, tail_hbm, out_hbm, rbufs, pbufs, wbufs, gsems, osems):
        wid = lax.axis_index("s") * NC + lax.axis_index("c")
        lanes = lax.iota(jnp.int32, 16)
        pitch = EMB + 1
        c65 = [(lanes + q * 16) * pitch for q in range(BLK // 16)]
        eq16 = [lanes + q * 16 for q in range(EMB // 16)]

        n_k = (nblk - wid + NW - 1) // NW

        def transpose(b, bvec):
            @plsc.parallel_loop(0, EMB, unroll=8)
            def _t1(e):
                for q in range(BLK // 16):
                    v = rbufs[b, e, pl.ds(q * 16, 16)]
                    plsc.store_scatter(pbufs, [bvec, c65[q] + e], v)

            @plsc.parallel_loop(0, BLK, unroll=8)
            def _t2(c):
                for q in range(EMB // 16):
                    v = plsc.load_gather(pbufs, [bvec, eq16[q] + c * pitch])
                    wbufs[b, pl.ds(c * EMB + q * 16, 16)] = v

        if tail:
            @pl.when(wid == NW - 1)
            def _tail():
                pltpu.sync_copy(tail_hbm, wbufs.at[0, pl.ds(0, tail * EMB)])
                pltpu.sync_copy(
                    wbufs.at[0, pl.ds(0, tail * EMB)],
                    out_hbm.at[pl.ds(nblk * BLK * EMB, tail * EMB)],
                )

        for b in range(NBUF):
            pltpu.async_copy(
                tab_hbm.at[:, pl.ds((wid + NW * b) * BLK, BLK)],
                rbufs.at[b],
                gsems.at[b],
            )

        def step(k, _):
            b = lax.rem(k, NBUF)
            blk = wid + NW * k
            bvec = jnp.full((16,), b, jnp.int32)

            @pl.when(k >= NBUF)
            def _wait_out():
                pltpu.make_async_copy(
                    wbufs.at[b], out_hbm.at[pl.ds(0, BLK * EMB)], osems.at[b]
                ).wait()

            pltpu.make_async_copy(
                tab_hbm.at[:, pl.ds(blk * BLK, BLK)],
                rbufs.at[b],
                gsems.at[b],
            ).wait()
            transpose(b, bvec)
            pltpu.async_copy(
                wbufs.at[b], out_hbm.at[pl.ds(blk * BLK * EMB, BLK * EMB)],
                osems.at[b],
            )

            @pl.when(k + NBUF < n_k)
            def _refill():
                nblk2 = wid + NW * (k + NBUF)
                pltpu.async_copy(
                    tab_hbm.at[:, pl.ds(nblk2 * BLK, BLK)],
                    rbufs.at[b],
                    gsems.at[b],
                )

            return _

        lax.fori_loop(0, n_k, step, None)

        for b in range(NBUF):
            pltpu.make_async_copy(
                wbufs.at[b], out_hbm.at[pl.ds(0, BLK * EMB)], osems.at[b]
            ).wait()

    return relayout_kernel


def _make_gather(seq: int):
    et = EMB // 8

    @functools.partial(
        pl.kernel,
        mesh=_mesh,
        out_type=jax.ShapeDtypeStruct((seq, et, NW, 8, BLK), jnp.float32),
        scratch_types=[
            pltpu.VMEM((seq, BLK), jnp.int32),
            pltpu.VMEM((NBUF, BLK, EMB), jnp.float32),
            pltpu.VMEM((NBUF, et, 8, BLK + 1), jnp.float32),
            pltpu.SemaphoreType.DMA((NBUF,)),
            pltpu.SemaphoreType.DMA((NBUF,)),
        ],
        compiler_params=pltpu.CompilerParams(
            use_tc_tiling_on_sc=False, needs_layout_passes=False
        ),
    )
    def gather_kernel(ids_hbm, table_hbm, out_hbm, idx_v, rbufs, tbufs, gsems, osems):
        wid = lax.axis_index("s") * NC + lax.axis_index("c")

        pltpu.sync_copy(ids_hbm.at[:, pl.ds(wid * BLK, BLK)], idx_v)

        lanes = lax.iota(jnp.int32, 16)
        e8vec = [(lanes + q * 16) // 8 for q in range(EMB // 16)]
        elvec = [(lanes + q * 16) % 8 for q in range(EMB // 16)]

        for b in range(NBUF):
            pltpu.async_copy(table_hbm.at[idx_v.at[b]], rbufs.at[b], gsems.at[b])

        def step(i, _):
            b = lax.rem(i, NBUF)

            @pl.when(i >= NBUF)
            def _wait_out():
                pltpu.make_async_copy(
                    tbufs.at[b, :, :, pl.ds(0, BLK)],
                    out_hbm.at[0, :, wid],
                    osems.at[b],
                ).wait()

            pltpu.make_async_copy(
                table_hbm.at[idx_v.at[i]], rbufs.at[b], gsems.at[b]
            ).wait()

            bvec = jnp.full((16,), b, jnp.int32)

            @plsc.parallel_loop(0, BLK, unroll=8)
            def _transpose(r):
                rvec = jnp.full((16,), r, jnp.int32)
                for q in range(EMB // 16):
                    v = rbufs[b, r, pl.ds(q * 16, 16)]
                    plsc.store_scatter(tbufs, [bvec, e8vec[q], elvec[q], rvec], v)

            pltpu.async_copy(
                tbufs.at[b, :, :, pl.ds(0, BLK)], out_hbm.at[i, :, wid], osems.at[b]
            )

            @pl.when(i + NBUF < seq)
            def _refill():
                pltpu.async_copy(
                    table_hbm.at[idx_v.at[i + NBUF]], rbufs.at[b], gsems.at[b]
                )

            return _

        lax.fori_loop(0, seq, step, None)

        for b in range(NBUF):
            pltpu.make_async_copy(
                tbufs.at[b, :, :, pl.ds(0, BLK)],
                out_hbm.at[0, :, wid],
                osems.at[b],
            ).wait()

    return gather_kernel


def kernel(input_ids, token_table):
    batch, seq = input_ids.shape
    vocab, _ = token_table.shape
    ids_t = jnp.transpose(input_ids.astype(jnp.int32))
    table_t = jnp.transpose(token_table)
    nblk = vocab // BLK
    tail_flat = jnp.reshape(
        lax.slice(token_table, (nblk * BLK, 0), (vocab, EMB)),
        ((vocab - nblk * BLK) * EMB,),
    )
    table_lin = jnp.reshape(
        _make_relayout(vocab)(table_t, tail_flat), (vocab, EMB)
    )
    out5 = _make_gather(seq)(ids_t, table_lin)
    return jnp.transpose(out5, (2, 4, 0, 1, 3)).reshape(batch, seq, EMB)


# speedup vs baseline: 1.8563x; 1.1332x over previous
import functools

import jax
import jax.numpy as jnp
from jax import lax
from jax.experimental import pallas as pl
from jax.experimental.pallas import tpu as pltpu
from jax.experimental.pallas import tpu_sc as plsc

EMB = 64
NC = 2
NS = 16
NW = NC * NS
BLK = 128
NBUF = 4

_mesh = plsc.VectorSubcoreMesh(core_axis_name="c", subcore_axis_name="s")


def _make_relayout(vocab: int):
    nblk = vocab // BLK
    tail = vocab - nblk * BLK

    @functools.partial(
        pl.kernel,
        mesh=_mesh,
        out_type=jax.ShapeDtypeStruct((vocab * EMB,), jnp.float32),
        scratch_types=[
            pltpu.VMEM((NBUF, EMB, BLK), jnp.float32),
            pltpu.VMEM((NBUF, BLK * (EMB + 1)), jnp.float32),
            pltpu.VMEM((NBUF, BLK * EMB), jnp.float32),
            pltpu.SemaphoreType.DMA((NBUF,)),
            pltpu.SemaphoreType.DMA((NBUF,)),
        ],
        compiler_params=pltpu.CompilerParams(needs_layout_passes=False),
    )
    def relayout_kernel(tab_hbm, tail_hbm, out_hbm, rbufs, pbufs, wbufs, gsems, osems):
        wid = lax.axis_index("s") * NC + lax.axis_index("c")
        lanes = lax.iota(jnp.int32, 16)
        pitch = EMB + 1
        c65 = [(lanes + q * 16) * pitch for q in range(BLK // 16)]
        eq16 = [lanes + q * 16 for q in range(EMB // 16)]

        n_k = (nblk - wid + NW - 1) // NW

        def transpose(b, bvec):
            @plsc.parallel_loop(0, EMB, unroll=16)
            def _t1(e):
                for q in range(BLK // 16):
                    v = rbufs[b, e, pl.ds(q * 16, 16)]
                    plsc.store_scatter(pbufs, [bvec, c65[q] + e], v)

            @plsc.parallel_loop(0, BLK, unroll=16)
            def _t2(c):
                for q in range(EMB // 16):
                    v = plsc.load_gather(pbufs, [bvec, eq16[q] + c * pitch])
                    wbufs[b, pl.ds(c * EMB + q * 16, 16)] = v

        if tail:
            @pl.when(wid == NW - 1)
            def _tail():
                pltpu.sync_copy(tail_hbm, wbufs.at[0, pl.ds(0, tail * EMB)])
                pltpu.sync_copy(
                    wbufs.at[0, pl.ds(0, tail * EMB)],
                    out_hbm.at[pl.ds(nblk * BLK * EMB, tail * EMB)],
                )

        for b in range(NBUF):
            pltpu.async_copy(
                tab_hbm.at[:, pl.ds((wid + NW * b) * BLK, BLK)],
                rbufs.at[b],
                gsems.at[b],
            )

        def step(k, _):
            b = lax.rem(k, NBUF)
            blk = wid + NW * k
            bvec = jnp.full((16,), b, jnp.int32)

            @pl.when(k >= NBUF)
            def _wait_out():
                pltpu.make_async_copy(
                    wbufs.at[b], out_hbm.at[pl.ds(0, BLK * EMB)], osems.at[b]
                ).wait()

            pltpu.make_async_copy(
                tab_hbm.at[:, pl.ds(blk * BLK, BLK)],
                rbufs.at[b],
                gsems.at[b],
            ).wait()
            transpose(b, bvec)
            pltpu.async_copy(
                wbufs.at[b], out_hbm.at[pl.ds(blk * BLK * EMB, BLK * EMB)],
                osems.at[b],
            )

            @pl.when(k + NBUF < n_k)
            def _refill():
                nblk2 = wid + NW * (k + NBUF)
                pltpu.async_copy(
                    tab_hbm.at[:, pl.ds(nblk2 * BLK, BLK)],
                    rbufs.at[b],
                    gsems.at[b],
                )

            return _

        lax.fori_loop(0, n_k, step, None)

        for b in range(NBUF):
            pltpu.make_async_copy(
                wbufs.at[b], out_hbm.at[pl.ds(0, BLK * EMB)], osems.at[b]
            ).wait()

    return relayout_kernel


def _make_gather(seq: int):
    et = EMB // 8

    @functools.partial(
        pl.kernel,
        mesh=_mesh,
        out_type=jax.ShapeDtypeStruct((seq, et, NW, 8, BLK), jnp.float32),
        scratch_types=[
            pltpu.VMEM((seq, BLK), jnp.int32),
            pltpu.VMEM((NBUF, BLK, EMB), jnp.float32),
            pltpu.VMEM((NBUF, et, 8, BLK + 1), jnp.float32),
            pltpu.SemaphoreType.DMA((NBUF,)),
            pltpu.SemaphoreType.DMA((NBUF,)),
        ],
        compiler_params=pltpu.CompilerParams(
            use_tc_tiling_on_sc=False, needs_layout_passes=False
        ),
    )
    def gather_kernel(ids_hbm, table_hbm, out_hbm, idx_v, rbufs, tbufs, gsems, osems):
        wid = lax.axis_index("s") * NC + lax.axis_index("c")

        pltpu.sync_copy(ids_hbm.at[:, pl.ds(wid * BLK, BLK)], idx_v)

        lanes = lax.iota(jnp.int32, 16)
        e8vec = [(lanes + q * 16) // 8 for q in range(EMB // 16)]
        elvec = [(lanes + q * 16) % 8 for q in range(EMB // 16)]

        for b in range(NBUF):
            pltpu.async_copy(table_hbm.at[idx_v.at[b]], rbufs.at[b], gsems.at[b])

        def step(i, _):
            b = lax.rem(i, NBUF)

            @pl.when(i >= NBUF)
            def _wait_out():
                pltpu.make_async_copy(
                    tbufs.at[b, :, :, pl.ds(0, BLK)],
                    out_hbm.at[0, :, wid],
                    osems.at[b],
                ).wait()

            pltpu.make_async_copy(
                table_hbm.at[idx_v.at[i]], rbufs.at[b], gsems.at[b]
            ).wait()

            bvec = jnp.full((16,), b, jnp.int32)

            @plsc.parallel_loop(0, BLK, unroll=8)
            def _transpose(r):
                rvec = jnp.full((16,), r, jnp.int32)
                for q in range(EMB // 16):
                    v = rbufs[b, r, pl.ds(q * 16, 16)]
                    plsc.store_scatter(tbufs, [bvec, e8vec[q], elvec[q], rvec], v)

            pltpu.async_copy(
                tbufs.at[b, :, :, pl.ds(0, BLK)], out_hbm.at[i, :, wid], osems.at[b]
            )

            @pl.when(i + NBUF < seq)
            def _refill():
                pltpu.async_copy(
                    table_hbm.at[idx_v.at[i + NBUF]], rbufs.at[b], gsems.at[b]
                )

            return _

        lax.fori_loop(0, seq, step, None)

        for b in range(NBUF):
            pltpu.make_async_copy(
                tbufs.at[b, :, :, pl.ds(0, BLK)],
                out_hbm.at[0, :, wid],
                osems.at[b],
            ).wait()

    return gather_kernel


def kernel(input_ids, token_table):
    batch, seq = input_ids.shape
    vocab, _ = token_table.shape
    ids_t = jnp.transpose(input_ids.astype(jnp.int32))
    table_t = jnp.transpose(token_table)
    nblk = vocab // BLK
    tail_flat = jnp.reshape(
        lax.slice(token_table, (nblk * BLK, 0), (vocab, EMB)),
        ((vocab - nblk * BLK) * EMB,),
    )
    table_lin = jnp.reshape(
        _make_relayout(vocab)(table_t, tail_flat), (vocab, EMB)
    )
    out5 = _make_gather(seq)(ids_t, table_lin)
    return jnp.transpose(out5, (2, 4, 0, 1, 3)).reshape(batch, seq, EMB)
